# baseline (speedup 1.0000x reference)
import functools

import jax
import jax.numpy as jnp
from jax import lax
from jax.experimental import pallas as pl
from jax.experimental.pallas import tpu as pltpu
from jax.experimental.pallas import tpu_sc as plsc

B = 16384
D = 128
L = 16
NSUB = D // L

NC = 2
NS = 16
NW = NC * NS
BPW = B // NW

CH = 128
NCH = BPW // CH
HH = CH // 2


def _transh_body(head_hbm, rel_hbm, tail_hbm, ent_hbm, rele_hbm, relh_hbm,
                 out_hbm, hidx, tidx, ridx, hbuf, tbuf, wbuf, rbuf, obuf, sem):
    cid = lax.axis_index("c")
    sid = lax.axis_index("s")
    wid = sid * NC + cid
    base = wid * BPW

    pltpu.sync_copy(head_hbm.at[pl.ds(base, BPW)], hidx)
    pltpu.sync_copy(tail_hbm.at[pl.ds(base, BPW)], tidx)
    pltpu.sync_copy(rel_hbm.at[pl.ds(base, BPW)], ridx)

    def row(i, rcarry):
        acc = jnp.zeros((L,), jnp.float32)
        hmts = []
        ws = []
        for j in range(NSUB):
            csl = pl.ds(j * L, L)
            h = hbuf[i, csl]
            t = tbuf[i, csl]
            w = wbuf[i, csl]
            hmt = h - t
            acc = acc + hmt * w
            hmts.append(hmt)
            ws.append(w)
        d = jnp.sum(acc)
        for j in range(NSUB):
            csl = pl.ds(j * L, L)
            r = rbuf[i, csl]
            obuf[i, csl] = hmts[j] + r - ws[j] * d
        return rcarry

    def chunk(c, carry):
        cps = [None, None]
        with jax.named_scope("issue"):
            for hf in (0, 1):
                isl = pl.ds(c * CH + hf * HH, HH)
                dsl = pl.ds(hf * HH, HH)
                cps[hf] = (
                    pltpu.async_copy(ent_hbm.at[hidx.at[isl]],
                                     hbuf.at[dsl], sem.at[hf]),
                    pltpu.async_copy(ent_hbm.at[tidx.at[isl]],
                                     tbuf.at[dsl], sem.at[hf]),
                    pltpu.async_copy(relh_hbm.at[ridx.at[isl]],
                                     wbuf.at[dsl], sem.at[hf]),
                    pltpu.async_copy(rele_hbm.at[ridx.at[isl]],
                                     rbuf.at[dsl], sem.at[hf]),
                )
        for hf in (0, 1):
            with jax.named_scope("wait%d" % hf):
                for cp in cps[hf]:
                    cp.wait()
            with jax.named_scope("comp%d" % hf):
                lax.fori_loop(hf * HH, (hf + 1) * HH, row, 0)
        with jax.named_scope("outcp"):
            pltpu.sync_copy(obuf, out_hbm.at[pl.ds(base + c * CH, CH)])
        return carry

    lax.fori_loop(0, NCH, chunk, 0)


_transh = functools.partial(
    pl.kernel,
    out_type=jax.ShapeDtypeStruct((B, D), jnp.float32),
    mesh=plsc.VectorSubcoreMesh(core_axis_name="c", subcore_axis_name="s"),
    compiler_params=pltpu.CompilerParams(needs_layout_passes=False),
    scratch_types=[
        pltpu.VMEM((BPW,), jnp.int32),
        pltpu.VMEM((BPW,), jnp.int32),
        pltpu.VMEM((BPW,), jnp.int32),
        pltpu.VMEM((CH, D), jnp.float32),
        pltpu.VMEM((CH, D), jnp.float32),
        pltpu.VMEM((CH, D), jnp.float32),
        pltpu.VMEM((CH, D), jnp.float32),
        pltpu.VMEM((CH, D), jnp.float32),
        pltpu.SemaphoreType.DMA((2,)),

    ],
)(_transh_body)


def kernel(head, relation, tail, ent_emb, rel_emb, rel_hyper):
    return _transh(head, relation, tail, ent_emb, rel_emb, rel_hyper)


# speedup vs baseline: 1.1928x; 1.1928x over previous
import functools

import jax
import jax.numpy as jnp
from jax import lax
from jax.experimental import pallas as pl
from jax.experimental.pallas import tpu as pltpu
from jax.experimental.pallas import tpu_sc as plsc

B = 16384
D = 128
L = 16
NSUB = D // L

NC = 2
NS = 16
NW = NC * NS
BPW = B // NW

CH = 64
NCH = BPW // CH
NPAIR = NCH // 2


def _transh_body(head_hbm, rel_hbm, tail_hbm, ent_hbm, rele_hbm, relh_hbm,
                 out_hbm, hidx, tidx, ridx, hbuf, tbuf, wbuf, rbuf, obuf,
                 gsem, osem):
    cid = lax.axis_index("c")
    sid = lax.axis_index("s")
    wid = sid * NC + cid
    base = wid * BPW

    pltpu.sync_copy(head_hbm.at[pl.ds(base, BPW)], hidx)
    pltpu.sync_copy(tail_hbm.at[pl.ds(base, BPW)], tidx)
    pltpu.sync_copy(rel_hbm.at[pl.ds(base, BPW)], ridx)

    def issue(c, p):
        isl = pl.ds(c * CH, CH)
        pltpu.async_copy(ent_hbm.at[hidx.at[isl]], hbuf.at[p], gsem.at[p])
        pltpu.async_copy(ent_hbm.at[tidx.at[isl]], tbuf.at[p], gsem.at[p])
        pltpu.async_copy(relh_hbm.at[ridx.at[isl]], wbuf.at[p], gsem.at[p])
        pltpu.async_copy(rele_hbm.at[ridx.at[isl]], rbuf.at[p], gsem.at[p])

    def drain_gathers(p):
        for buf in (hbuf, tbuf, wbuf, rbuf):
            pltpu.make_async_copy(
                ent_hbm.at[pl.ds(0, CH)], buf.at[p], gsem.at[p]).wait()

    def drain_out(p):
        pltpu.make_async_copy(
            obuf.at[p], out_hbm.at[pl.ds(0, CH)], osem.at[p]).wait()

    def compute(p):
        def row(i, rcarry):
            acc = jnp.zeros((L,), jnp.float32)
            hmts = []
            ws = []
            for j in range(NSUB):
                csl = pl.ds(j * L, L)
                h = hbuf[p, i, csl]
                t = tbuf[p, i, csl]
                w = wbuf[p, i, csl]
                hmt = h - t
                acc = acc + hmt * w
                hmts.append(hmt)
                ws.append(w)
            d = jnp.sum(acc)
            for j in range(NSUB):
                csl = pl.ds(j * L, L)
                r = rbuf[p, i, csl]
                obuf[p, i, csl] = hmts[j] + r - ws[j] * d
            return rcarry

        lax.fori_loop(0, CH, row, 0)

    issue(0, 0)
    issue(1, 1)

    def pair(g, carry):
        for p in (0, 1):
            c = 2 * g + p
            drain_gathers(p)

            @pl.when(g > 0)
            def _drain_prev_out():
                drain_out(p)

            compute(p)

            @pl.when(c + 2 < NCH)
            def _issue_next():
                issue(c + 2, p)

            pltpu.async_copy(
                obuf.at[p], out_hbm.at[pl.ds(base + c * CH, CH)], osem.at[p])
        return carry

    lax.fori_loop(0, NPAIR, pair, 0)
    drain_out(0)
    drain_out(1)


_transh = functools.partial(
    pl.kernel,
    out_type=jax.ShapeDtypeStruct((B, D), jnp.float32),
    mesh=plsc.VectorSubcoreMesh(core_axis_name="c", subcore_axis_name="s"),
    compiler_params=pltpu.CompilerParams(needs_layout_passes=False),
    scratch_types=[
        pltpu.VMEM((BPW,), jnp.int32),
        pltpu.VMEM((BPW,), jnp.int32),
        pltpu.VMEM((BPW,), jnp.int32),
        pltpu.VMEM((2, CH, D), jnp.float32),
        pltpu.VMEM((2, CH, D), jnp.float32),
        pltpu.VMEM((2, CH, D), jnp.float32),
        pltpu.VMEM((2, CH, D), jnp.float32),
        pltpu.VMEM((2, CH, D), jnp.float32),
        pltpu.SemaphoreType.DMA((2,)),
        pltpu.SemaphoreType.DMA((2,)),
    ],
)(_transh_body)


def kernel(head, relation, tail, ent_emb, rel_emb, rel_hyper):
    return _transh(head, relation, tail, ent_emb, rel_emb, rel_hyper)


# speedup vs baseline: 1.2691x; 1.0639x over previous
import functools

import jax
import jax.numpy as jnp
from jax import lax
from jax.experimental import pallas as pl
from jax.experimental.pallas import tpu as pltpu
from jax.experimental.pallas import tpu_sc as plsc

B = 16384
D = 128
L = 16
NSUB = D // L

NC = 2
NS = 16
NW = NC * NS
BPW = B // NW

CH = 64
NCH = BPW // CH
NPAIR = NCH // 2


def _transh_body(head_hbm, rel_hbm, tail_hbm, ent_hbm, relwr_hbm,
                 out_hbm, hidx, tidx, ridx, hbuf, tbuf, wrbuf, obuf,
                 gsem, osem):
    cid = lax.axis_index("c")
    sid = lax.axis_index("s")
    wid = sid * NC + cid
    base = wid * BPW

    pltpu.sync_copy(head_hbm.at[pl.ds(base, BPW)], hidx)
    pltpu.sync_copy(tail_hbm.at[pl.ds(base, BPW)], tidx)
    pltpu.sync_copy(rel_hbm.at[pl.ds(base, BPW)], ridx)

    def issue(c, p):
        isl = pl.ds(c * CH, CH)
        pltpu.async_copy(ent_hbm.at[hidx.at[isl]], hbuf.at[p], gsem.at[p])
        pltpu.async_copy(ent_hbm.at[tidx.at[isl]], tbuf.at[p], gsem.at[p])
        pltpu.async_copy(relwr_hbm.at[ridx.at[isl]], wrbuf.at[p], gsem.at[p])

    def drain_gathers(p):
        for src, buf in ((ent_hbm, hbuf), (ent_hbm, tbuf),
                         (relwr_hbm, wrbuf)):
            pltpu.make_async_copy(
                src.at[pl.ds(0, CH)], buf.at[p], gsem.at[p]).wait()

    def drain_out(p):
        pltpu.make_async_copy(
            obuf.at[p], out_hbm.at[pl.ds(0, CH)], osem.at[p]).wait()

    def compute(p):
        def row(i, rcarry):
            acc = jnp.zeros((L,), jnp.float32)
            hmts = []
            ws = []
            for j in range(NSUB // 2):
                w2 = plsc.bitcast(wrbuf[p, i, pl.ds(j * L, L)], jnp.bfloat16)
                for s, w in enumerate(
                        plsc.unpack(w2, format=plsc.PackFormat.INTERLEAVED)):
                    csl = pl.ds((2 * j + s) * L, L)
                    h = hbuf[p, i, csl]
                    t = tbuf[p, i, csl]
                    hmt = h - t
                    acc = acc + hmt * w
                    hmts.append(hmt)
                    ws.append(w)
            d = jnp.sum(acc)
            for j in range(NSUB // 2):
                r2 = plsc.bitcast(
                    wrbuf[p, i, pl.ds(D // 2 + j * L, L)], jnp.bfloat16)
                for s, r in enumerate(
                        plsc.unpack(r2, format=plsc.PackFormat.INTERLEAVED)):
                    k = 2 * j + s
                    obuf[p, i, pl.ds(k * L, L)] = hmts[k] + r - ws[k] * d
            return rcarry

        lax.fori_loop(0, CH, row, 0)

    issue(0, 0)
    issue(1, 1)

    def pair(g, carry):
        for p in (0, 1):
            c = 2 * g + p
            drain_gathers(p)

            @pl.when(g > 0)
            def _drain_prev_out():
                drain_out(p)

            compute(p)

            @pl.when(c + 2 < NCH)
            def _issue_next():
                issue(c + 2, p)

            pltpu.async_copy(
                obuf.at[p], out_hbm.at[pl.ds(base + c * CH, CH)], osem.at[p])
        return carry

    lax.fori_loop(0, NPAIR, pair, 0)
    drain_out(0)
    drain_out(1)


_transh = functools.partial(
    pl.kernel,
    out_type=jax.ShapeDtypeStruct((B, D), jnp.float32),
    mesh=plsc.VectorSubcoreMesh(core_axis_name="c", subcore_axis_name="s"),
    compiler_params=pltpu.CompilerParams(needs_layout_passes=False),
    scratch_types=[
        pltpu.VMEM((BPW,), jnp.int32),
        pltpu.VMEM((BPW,), jnp.int32),
        pltpu.VMEM((BPW,), jnp.int32),
        pltpu.VMEM((2, CH, D), jnp.float32),
        pltpu.VMEM((2, CH, D), jnp.float32),
        pltpu.VMEM((2, CH, D), jnp.int32),
        pltpu.VMEM((2, CH, D), jnp.float32),
        pltpu.SemaphoreType.DMA((2,)),
        pltpu.SemaphoreType.DMA((2,)),
    ],
)(_transh_body)


def _interleave_bf16(x):
    n = x.shape[0]
    xi = (x.astype(jnp.bfloat16)
          .reshape(n, D // (2 * L), 2, L).swapaxes(2, 3).reshape(n, D // 2, 2))
    return lax.bitcast_convert_type(xi, jnp.int32)


def kernel(head, relation, tail, ent_emb, rel_emb, rel_hyper):
    relwr = jnp.concatenate(
        [_interleave_bf16(rel_hyper), _interleave_bf16(rel_emb)], axis=1)
    return _transh(head, relation, tail, ent_emb, relwr)
